# baseline (speedup 1.0000x reference)
import functools

import jax
import jax.numpy as jnp
from jax import lax
from jax.experimental import pallas as pl
from jax.experimental.pallas import tpu as pltpu

_ROWS = 256
_NB = 8
_D = 4


def _append_body(seq, tail, nc, kc, vc, ko, vo, ok, ov, buf, tk, tv,
                 rsem, wsem, trsem, twsem):
    h = kc.shape[1]
    per_t = h * nc
    n = 2 * per_t

    def chunk_coords(i):
        t = i // per_t
        r = lax.rem(i, per_t)
        head = r // nc
        row0 = lax.rem(r, nc) * _ROWS
        return t, head, row0

    def read_copy(i, slot):
        t, head, row0 = chunk_coords(i)

        @pl.when(t == 0)
        def _():
            pltpu.make_async_copy(
                kc.at[0, head, pl.ds(row0, _ROWS)], buf.at[slot], rsem.at[slot]
            ).start()

        @pl.when(t != 0)
        def _():
            pltpu.make_async_copy(
                vc.at[0, head, pl.ds(row0, _ROWS)], buf.at[slot], rsem.at[slot]
            ).start()

    def wait_read(i, slot):
        t, head, row0 = chunk_coords(i)
        pltpu.make_async_copy(
            kc.at[0, head, pl.ds(row0, _ROWS)], buf.at[slot], rsem.at[slot]
        ).wait()

    def write_copy(i, slot):
        t, head, row0 = chunk_coords(i)

        @pl.when(t == 0)
        def _():
            pltpu.make_async_copy(
                buf.at[slot], ok.at[0, head, pl.ds(row0, _ROWS)], wsem.at[slot]
            ).start()

        @pl.when(t != 0)
        def _():
            pltpu.make_async_copy(
                buf.at[slot], ov.at[0, head, pl.ds(row0, _ROWS)], wsem.at[slot]
            ).start()

    def wait_write(i, slot):
        t, head, row0 = chunk_coords(i)
        pltpu.make_async_copy(
            buf.at[slot], ok.at[0, head, pl.ds(row0, _ROWS)], wsem.at[slot]
        ).wait()

    pltpu.make_async_copy(ko, tk, trsem.at[0]).start()
    pltpu.make_async_copy(vo, tv, trsem.at[1]).start()

    def step(i, _):
        @pl.when(i < n)
        def _():
            slot = lax.rem(i, _NB)

            @pl.when(i >= _NB)
            def _():
                wait_write(i - _NB, slot)

            read_copy(i, slot)

        k = i - _D

        @pl.when(k >= 0)
        def _():
            slot = lax.rem(k, _NB)
            wait_read(k, slot)
            write_copy(k, slot)

        return 0

    lax.fori_loop(0, n + _D, step, 0)

    def drain(j, _):
        i = n - _NB + j
        wait_write(i, lax.rem(i, _NB))
        return 0

    lax.fori_loop(0, _NB, drain, 0)

    pltpu.make_async_copy(ko, tk, trsem.at[0]).wait()
    pltpu.make_async_copy(vo, tv, trsem.at[1]).wait()
    for head in range(h):
        pltpu.make_async_copy(
            tk.at[0, head], ok.at[0, head, pl.ds(seq, tail)], twsem.at[0]
        ).start()
        pltpu.make_async_copy(
            tv.at[0, head], ov.at[0, head, pl.ds(seq, tail)], twsem.at[1]
        ).start()
    for head in range(h):
        pltpu.make_async_copy(
            tk.at[0, head], ok.at[0, head, pl.ds(seq, tail)], twsem.at[0]
        ).wait()
        pltpu.make_async_copy(
            tv.at[0, head], ov.at[0, head, pl.ds(seq, tail)], twsem.at[1]
        ).wait()


def kernel(k_cache, v_cache, k_out, v_out):
    b, h, seq, n, d = k_cache.shape
    tail = k_out.shape[2]
    dtype = k_cache.dtype
    kc, vc, ko, vo = (
        lax.bitcast_convert_type(x, jnp.bfloat16)
        for x in (k_cache, v_cache, k_out, v_out)
    )
    rows = _ROWS
    assert seq % rows == 0
    nc = seq // rows
    out_sds = jax.ShapeDtypeStruct((b, h, seq + tail, n, d), jnp.bfloat16)

    fn = pl.pallas_call(
        functools.partial(_append_body, seq, tail, nc),
        in_specs=[pl.BlockSpec(memory_space=pl.ANY)] * 4,
        out_specs=[pl.BlockSpec(memory_space=pl.ANY)] * 2,
        out_shape=[out_sds, out_sds],
        scratch_shapes=[
            pltpu.VMEM((_NB, rows, n, d), jnp.bfloat16),
            pltpu.VMEM((b, h, tail, n, d), jnp.bfloat16),
            pltpu.VMEM((b, h, tail, n, d), jnp.bfloat16),
            pltpu.SemaphoreType.DMA((_NB,)),
            pltpu.SemaphoreType.DMA((_NB,)),
            pltpu.SemaphoreType.DMA((2,)),
            pltpu.SemaphoreType.DMA((2,)),
        ],
    )
    k_new, v_new = fn(kc, vc, ko, vo)
    return (
        lax.bitcast_convert_type(k_new, dtype),
        lax.bitcast_convert_type(v_new, dtype),
    )


# speedup vs baseline: 2.4029x; 2.4029x over previous
import functools

import jax
import jax.numpy as jnp
from jax import lax
from jax.experimental import pallas as pl
from jax.experimental.pallas import tpu as pltpu
from jax.experimental.pallas import tpu_sc as plsc

_CH = 16
_NW = 32


def _sc_body(seq, tail, kc, vc, ko, vo, ok, ov, ba, bb, sra, srb, swa, swb):
    c = lax.axis_index("c")
    s = lax.axis_index("s")
    w = s * 2 + c
    nq = _NW // 8
    head = w // nq
    q = lax.rem(w, nq)
    rows_w = seq // nq
    base = q * rows_w
    n = rows_w // _CH

    def stream(src, dst):
        def rd(i, buf, sem):
            return pltpu.make_async_copy(
                src.at[0, head, pl.ds(base + i * _CH, _CH)], buf, sem
            )

        def wr(i, buf, sem):
            return pltpu.make_async_copy(
                buf, dst.at[0, head, pl.ds(base + i * _CH, _CH)], sem
            )

        rd(0, ba, sra).start()
        rd(1, bb, srb).start()

        def step(j, carry):
            i = 2 * j
            rd(i, ba, sra).wait()
            wr(i, ba, swa).start()
            rd(i + 1, bb, srb).wait()
            wr(i + 1, bb, swb).start()
            wr(i, ba, swa).wait()
            rd(i + 2, ba, sra).start()
            wr(i + 1, bb, swb).wait()
            rd(i + 3, bb, srb).start()
            return carry

        lax.fori_loop(0, n // 2 - 1, step, 0)

        i = n - 2
        rd(i, ba, sra).wait()
        wr(i, ba, swa).start()
        rd(i + 1, bb, srb).wait()
        wr(i + 1, bb, swb).start()
        wr(i, ba, swa).wait()
        wr(i + 1, bb, swb).wait()

    stream(kc, ok)
    stream(vc, ov)

    @pl.when(q == 0)
    def _k_tail():
        pltpu.make_async_copy(ko.at[0, head], ba, sra).start()
        pltpu.make_async_copy(ko.at[0, head], ba, sra).wait()
        pltpu.make_async_copy(ba, ok.at[0, head, pl.ds(seq, tail)], swa).start()
        pltpu.make_async_copy(ba, ok.at[0, head, pl.ds(seq, tail)], swa).wait()

    @pl.when(q == 1)
    def _v_tail():
        pltpu.make_async_copy(vo.at[0, head], bb, srb).start()
        pltpu.make_async_copy(vo.at[0, head], bb, srb).wait()
        pltpu.make_async_copy(bb, ov.at[0, head, pl.ds(seq, tail)], swb).start()
        pltpu.make_async_copy(bb, ov.at[0, head, pl.ds(seq, tail)], swb).wait()


def kernel(k_cache, v_cache, k_out, v_out):
    b, h, seq, n, d = k_cache.shape
    tail = k_out.shape[2]
    assert tail == _CH and seq % (2 * _CH * (_NW // 8)) == 0 and b * h == 8
    out_sds = jax.ShapeDtypeStruct((b, h, seq + tail, n, d), k_cache.dtype)
    mesh = plsc.VectorSubcoreMesh(core_axis_name="c", subcore_axis_name="s")
    fn = pl.kernel(
        functools.partial(_sc_body, seq, tail),
        mesh=mesh,
        out_type=[out_sds, out_sds],
        scratch_types=[
            pltpu.VMEM((_CH, n, d), k_cache.dtype),
            pltpu.VMEM((_CH, n, d), k_cache.dtype),
            pltpu.SemaphoreType.DMA,
            pltpu.SemaphoreType.DMA,
            pltpu.SemaphoreType.DMA,
            pltpu.SemaphoreType.DMA,
        ],
    )
    k_new, v_new = fn(k_cache, v_cache, k_out, v_out)
    return (k_new, v_new)


# speedup vs baseline: 2.4386x; 1.0149x over previous
import functools

import jax
import jax.numpy as jnp
from jax import lax
from jax.experimental import pallas as pl
from jax.experimental.pallas import tpu as pltpu
from jax.experimental.pallas import tpu_sc as plsc

_CH = 8
_NB = 6
_D = 3
_NW = 32


def _sc_body(seq, tail, kc, vc, ko, vo, ok, ov, *rest):
    bufs = rest[:_NB]
    rsem = rest[_NB:2 * _NB]
    wsem = rest[2 * _NB:3 * _NB]
    c = lax.axis_index("c")
    s = lax.axis_index("s")
    w = s * 2 + c
    nq = _NW // 8
    head = w // nq
    q = lax.rem(w, nq)
    rows_w = seq // nq
    base = q * rows_w
    n = rows_w // _CH

    def stream(src, dst):
        def rd(i, u):
            return pltpu.make_async_copy(
                src.at[0, head, pl.ds(base + i * _CH, _CH)], bufs[u], rsem[u]
            )

        def wr(i, u):
            return pltpu.make_async_copy(
                bufs[u], dst.at[0, head, pl.ds(base + i * _CH, _CH)], wsem[u]
            )

        for i in range(_D):
            rd(i, i).start()
        for i in range(_NB):
            if i >= _D:
                wr(i - _D, i - _D).wait()
            if i + _D < n:
                rd(i + _D, (i + _D) % _NB).start()
            rd(i, i).wait()
            wr(i, i).start()

        n_steady = ((n - _NB - _D) // _NB) * _NB

        def step(j, carry):
            i0 = _NB + j * _NB
            for uu in range(_NB):
                i = i0 + uu
                wr(i - _D, (uu - _D) % _NB).wait()
                rd(i + _D, (uu + _D) % _NB).start()
                rd(i, uu).wait()
                wr(i, uu).start()
            return carry

        lax.fori_loop(0, n_steady // _NB, step, 0)

        for i in range(_NB + n_steady, n):
            wr(i - _D, (i - _D) % _NB).wait()
            if i + _D < n:
                rd(i + _D, (i + _D) % _NB).start()
            rd(i, i % _NB).wait()
            wr(i, i % _NB).start()
        for i in range(n - _D, n):
            wr(i, i % _NB).wait()

    stream(kc, ok)
    stream(vc, ov)

    @pl.when(q == 0)
    def _k_tail():
        for j in range(tail // _CH):
            pltpu.make_async_copy(
                ko.at[0, head, pl.ds(j * _CH, _CH)], bufs[j], rsem[j]
            ).start()
        for j in range(tail // _CH):
            pltpu.make_async_copy(
                ko.at[0, head, pl.ds(j * _CH, _CH)], bufs[j], rsem[j]
            ).wait()
            pltpu.make_async_copy(
                bufs[j], ok.at[0, head, pl.ds(seq + j * _CH, _CH)], wsem[j]
            ).start()
        for j in range(tail // _CH):
            pltpu.make_async_copy(
                bufs[j], ok.at[0, head, pl.ds(seq + j * _CH, _CH)], wsem[j]
            ).wait()

    @pl.when(q == 1)
    def _v_tail():
        for j in range(tail // _CH):
            pltpu.make_async_copy(
                vo.at[0, head, pl.ds(j * _CH, _CH)], bufs[j], rsem[j]
            ).start()
        for j in range(tail // _CH):
            pltpu.make_async_copy(
                vo.at[0, head, pl.ds(j * _CH, _CH)], bufs[j], rsem[j]
            ).wait()
            pltpu.make_async_copy(
                bufs[j], ov.at[0, head, pl.ds(seq + j * _CH, _CH)], wsem[j]
            ).start()
        for j in range(tail // _CH):
            pltpu.make_async_copy(
                bufs[j], ov.at[0, head, pl.ds(seq + j * _CH, _CH)], wsem[j]
            ).wait()


def kernel(k_cache, v_cache, k_out, v_out):
    b, h, seq, n, d = k_cache.shape
    tail = k_out.shape[2]
    nq = _NW // 8
    rows_w = seq // nq
    assert b * h == 8 and seq % nq == 0 and rows_w % _CH == 0
    assert rows_w // _CH >= _NB + 2 * _D
    assert tail % _CH == 0 and tail // _CH <= _NB
    out_sds = jax.ShapeDtypeStruct((b, h, seq + tail, n, d), k_cache.dtype)
    mesh = plsc.VectorSubcoreMesh(core_axis_name="c", subcore_axis_name="s")
    fn = pl.kernel(
        functools.partial(_sc_body, seq, tail),
        mesh=mesh,
        out_type=[out_sds, out_sds],
        scratch_types=(
            [pltpu.VMEM((_CH, n, d), k_cache.dtype) for _ in range(_NB)]
            + [pltpu.SemaphoreType.DMA] * (2 * _NB)
        ),
    )
    k_new, v_new = fn(k_cache, v_cache, k_out, v_out)
    return (k_new, v_new)


# speedup vs baseline: 2.4423x; 1.0015x over previous
import functools

import jax
import jax.numpy as jnp
from jax import lax
from jax.experimental import pallas as pl
from jax.experimental.pallas import tpu as pltpu
from jax.experimental.pallas import tpu_sc as plsc

_CH = 4
_NB = 12
_D = 6
_NW = 32


def _sc_body(seq, tail, kc, vc, ko, vo, ok, ov, *rest):
    bufs = rest[:_NB]
    rsem = rest[_NB:2 * _NB]
    wsem = rest[2 * _NB:3 * _NB]
    c = lax.axis_index("c")
    s = lax.axis_index("s")
    w = s * 2 + c
    nq = _NW // 8
    head = w // nq
    q = lax.rem(w, nq)
    rows_w = seq // nq
    base = q * rows_w
    n = rows_w // _CH

    def stream(src, dst):
        def rd(i, u):
            return pltpu.make_async_copy(
                src.at[0, head, pl.ds(base + i * _CH, _CH)], bufs[u], rsem[u]
            )

        def wr(i, u):
            return pltpu.make_async_copy(
                bufs[u], dst.at[0, head, pl.ds(base + i * _CH, _CH)], wsem[u]
            )

        for i in range(_D):
            rd(i, i).start()
        for i in range(_NB):
            if i >= _D:
                wr(i - _D, i - _D).wait()
            if i + _D < n:
                rd(i + _D, (i + _D) % _NB).start()
            rd(i, i).wait()
            wr(i, i).start()

        n_steady = ((n - _NB - _D) // _NB) * _NB

        def step(j, carry):
            i0 = _NB + j * _NB
            for uu in range(_NB):
                i = i0 + uu
                wr(i - _D, (uu - _D) % _NB).wait()
                rd(i + _D, (uu + _D) % _NB).start()
                rd(i, uu).wait()
                wr(i, uu).start()
            return carry

        lax.fori_loop(0, n_steady // _NB, step, 0)

        for i in range(_NB + n_steady, n):
            wr(i - _D, (i - _D) % _NB).wait()
            if i + _D < n:
                rd(i + _D, (i + _D) % _NB).start()
            rd(i, i % _NB).wait()
            wr(i, i % _NB).start()
        for i in range(n - _D, n):
            wr(i, i % _NB).wait()

    stream(kc, ok)
    stream(vc, ov)

    @pl.when(q == 0)
    def _k_tail():
        for j in range(tail // _CH):
            pltpu.make_async_copy(
                ko.at[0, head, pl.ds(j * _CH, _CH)], bufs[j], rsem[j]
            ).start()
        for j in range(tail // _CH):
            pltpu.make_async_copy(
                ko.at[0, head, pl.ds(j * _CH, _CH)], bufs[j], rsem[j]
            ).wait()
            pltpu.make_async_copy(
                bufs[j], ok.at[0, head, pl.ds(seq + j * _CH, _CH)], wsem[j]
            ).start()
        for j in range(tail // _CH):
            pltpu.make_async_copy(
                bufs[j], ok.at[0, head, pl.ds(seq + j * _CH, _CH)], wsem[j]
            ).wait()

    @pl.when(q == 1)
    def _v_tail():
        for j in range(tail // _CH):
            pltpu.make_async_copy(
                vo.at[0, head, pl.ds(j * _CH, _CH)], bufs[j], rsem[j]
            ).start()
        for j in range(tail // _CH):
            pltpu.make_async_copy(
                vo.at[0, head, pl.ds(j * _CH, _CH)], bufs[j], rsem[j]
            ).wait()
            pltpu.make_async_copy(
                bufs[j], ov.at[0, head, pl.ds(seq + j * _CH, _CH)], wsem[j]
            ).start()
        for j in range(tail // _CH):
            pltpu.make_async_copy(
                bufs[j], ov.at[0, head, pl.ds(seq + j * _CH, _CH)], wsem[j]
            ).wait()


def kernel(k_cache, v_cache, k_out, v_out):
    b, h, seq, n, d = k_cache.shape
    tail = k_out.shape[2]
    nq = _NW // 8
    rows_w = seq // nq
    assert b * h == 8 and seq % nq == 0 and rows_w % _CH == 0
    assert rows_w // _CH >= _NB + 2 * _D
    assert tail % _CH == 0 and tail // _CH <= _NB
    out_sds = jax.ShapeDtypeStruct((b, h, seq + tail, n, d), k_cache.dtype)
    mesh = plsc.VectorSubcoreMesh(core_axis_name="c", subcore_axis_name="s")
    fn = pl.kernel(
        functools.partial(_sc_body, seq, tail),
        mesh=mesh,
        out_type=[out_sds, out_sds],
        scratch_types=(
            [pltpu.VMEM((_CH, n, d), k_cache.dtype) for _ in range(_NB)]
            + [pltpu.SemaphoreType.DMA] * (2 * _NB)
        ),
    )
    k_new, v_new = fn(k_cache, v_cache, k_out, v_out)
    return (k_new, v_new)


# speedup vs baseline: 2.6102x; 1.0688x over previous
import functools

import jax
import jax.numpy as jnp
from jax import lax
from jax.experimental import pallas as pl
from jax.experimental.pallas import tpu as pltpu
from jax.experimental.pallas import tpu_sc as plsc

_CH = 4
_NB = 12
_D = 6
_NW = 32


def _sc_body(seq, tail, kc, vc, ko, vo, ok, ov, *rest):
    shared = rest[0]
    rsem = rest[1:1 + _NB]
    wsem = rest[1 + _NB:1 + 2 * _NB]
    c = lax.axis_index("c")
    s = lax.axis_index("s")
    bufs = [shared.at[s, u] for u in range(_NB)]
    w = s * 2 + c
    nq = _NW // 8
    head = w // nq
    q = lax.rem(w, nq)
    rows_w = seq // nq
    base = q * rows_w
    n = rows_w // _CH

    def stream(src, dst):
        def rd(i, u):
            return pltpu.make_async_copy(
                src.at[0, head, pl.ds(base + i * _CH, _CH)], bufs[u], rsem[u]
            )

        def wr(i, u):
            return pltpu.make_async_copy(
                bufs[u], dst.at[0, head, pl.ds(base + i * _CH, _CH)], wsem[u]
            )

        for i in range(_D):
            rd(i, i).start()
        for i in range(_NB):
            if i >= _D:
                wr(i - _D, i - _D).wait()
            if i + _D < n:
                rd(i + _D, (i + _D) % _NB).start()
            rd(i, i).wait()
            wr(i, i).start()

        n_steady = ((n - _NB - _D) // _NB) * _NB

        def step(j, carry):
            i0 = _NB + j * _NB
            for uu in range(_NB):
                i = i0 + uu
                wr(i - _D, (uu - _D) % _NB).wait()
                rd(i + _D, (uu + _D) % _NB).start()
                rd(i, uu).wait()
                wr(i, uu).start()
            return carry

        lax.fori_loop(0, n_steady // _NB, step, 0)

        for i in range(_NB + n_steady, n):
            wr(i - _D, (i - _D) % _NB).wait()
            if i + _D < n:
                rd(i + _D, (i + _D) % _NB).start()
            rd(i, i % _NB).wait()
            wr(i, i % _NB).start()
        for i in range(n - _D, n):
            wr(i, i % _NB).wait()

    stream(kc, ok)
    stream(vc, ov)

    @pl.when(q == 0)
    def _k_tail():
        for j in range(tail // _CH):
            pltpu.make_async_copy(
                ko.at[0, head, pl.ds(j * _CH, _CH)], bufs[j], rsem[j]
            ).start()
        for j in range(tail // _CH):
            pltpu.make_async_copy(
                ko.at[0, head, pl.ds(j * _CH, _CH)], bufs[j], rsem[j]
            ).wait()
            pltpu.make_async_copy(
                bufs[j], ok.at[0, head, pl.ds(seq + j * _CH, _CH)], wsem[j]
            ).start()
        for j in range(tail // _CH):
            pltpu.make_async_copy(
                bufs[j], ok.at[0, head, pl.ds(seq + j * _CH, _CH)], wsem[j]
            ).wait()

    @pl.when(q == 1)
    def _v_tail():
        for j in range(tail // _CH):
            pltpu.make_async_copy(
                vo.at[0, head, pl.ds(j * _CH, _CH)], bufs[j], rsem[j]
            ).start()
        for j in range(tail // _CH):
            pltpu.make_async_copy(
                vo.at[0, head, pl.ds(j * _CH, _CH)], bufs[j], rsem[j]
            ).wait()
            pltpu.make_async_copy(
                bufs[j], ov.at[0, head, pl.ds(seq + j * _CH, _CH)], wsem[j]
            ).start()
        for j in range(tail // _CH):
            pltpu.make_async_copy(
                bufs[j], ov.at[0, head, pl.ds(seq + j * _CH, _CH)], wsem[j]
            ).wait()


def kernel(k_cache, v_cache, k_out, v_out):
    b, h, seq, n, d = k_cache.shape
    tail = k_out.shape[2]
    nq = _NW // 8
    rows_w = seq // nq
    assert b * h == 8 and seq % nq == 0 and rows_w % _CH == 0
    assert rows_w // _CH >= _NB + 2 * _D
    assert tail % _CH == 0 and tail // _CH <= _NB
    out_sds = jax.ShapeDtypeStruct((b, h, seq + tail, n, d), k_cache.dtype)
    mesh = plsc.VectorSubcoreMesh(core_axis_name="c", subcore_axis_name="s")
    fn = pl.kernel(
        functools.partial(_sc_body, seq, tail),
        mesh=mesh,
        out_type=[out_sds, out_sds],
        scratch_types=(
            [pltpu.VMEM_SHARED((16, _NB, _CH, n, d), k_cache.dtype)]
            + [pltpu.SemaphoreType.DMA] * (2 * _NB)
        ),
    )
    k_new, v_new = fn(k_cache, v_cache, k_out, v_out)
    return (k_new, v_new)
